# Initial kernel scaffold; baseline (speedup 1.0000x reference)
import jax
import jax.numpy as jnp
from jax.experimental import pallas as pl


def kernel(pred_norm, connectivity, coords_norm, prop_E_norm, prop_A_norm, prop_I22_norm, F_ext_norm, u_scale, theta_scale):
    raise NotImplementedError("write your pallas kernel here")


# speedup vs baseline: 95.5219x; 95.5219x over previous
import functools

import jax
import jax.numpy as jnp
from jax import lax
from jax.experimental import pallas as pl
from jax.experimental.pallas import tpu as pltpu
from jax.experimental.pallas import tpu_sc as plsc

_NC = 2
_NS = 16
_L = 16

_CHUNK = 2000
_SUB = _CHUNK // _L

_DRAIN = 18768


def _rsqrt(q):
  qi = lax.bitcast_convert_type(q, jnp.int32)
  yi = jnp.int32(0x5F3759DF) - (qi >> 1)
  y = lax.bitcast_convert_type(yi, jnp.float32)
  h = q * jnp.float32(0.5)
  y = y * (jnp.float32(1.5) - h * y * y)
  y = y * (jnp.float32(1.5) - h * y * y)
  return y


def _sc_edge_kernel(n_nodes, n_edges):
  epw = n_edges // (_NC * _NS)
  n_chunks = epw // _CHUNK
  n3 = 3 * n_nodes
  last = n3 - 15 * _DRAIN
  stage = 6256
  stage_last = n_nodes - 15 * stage

  mesh = plsc.VectorSubcoreMesh(core_axis_name="c", subcore_axis_name="s")

  def body(conn_hbm, pe_hbm, pa_hbm, pi_hbm, cols_hbm, out_hbm,
           tab_cx, tab_cz, tab_p0, tab_p1, tab_p2, acc,
           conn_v, idxa_v, idxb_v, ia0, ia1, ia2, ib0, ib1, ib2,
           pe_v, pa_v, pi_v,
           g_cxa, g_cza, g_cxb, g_czb, g_p0a, g_p1a, g_p2a, g_p0b, g_p1b,
           g_p2b, f_xa, f_ya, f_za, f_xb, f_yb, f_zb, dr_v,
           gsem, ssem):
    cid = lax.axis_index("c")
    sid = lax.axis_index("s")
    wid = cid * _NS + sid
    tabs = (tab_cx, tab_cz, tab_p0, tab_p1, tab_p2)

    def stage_tab(tab_i, ref, cnt, off):
      pltpu.sync_copy(cols_hbm.at[tab_i, pl.ds(off, cnt)],
                      dr_v.at[pl.ds(0, cnt)])
      pltpu.sync_copy(dr_v.at[pl.ds(0, cnt)], ref.at[pl.ds(off, cnt)])

    @pl.when(sid < _NS - 1)
    def _():
      for t, ref in enumerate(tabs):
        stage_tab(t, ref, stage, sid * stage)

    @pl.when(sid == _NS - 1)
    def _():
      for t, ref in enumerate(tabs):
        stage_tab(t, ref, stage_last, 15 * stage)

    def zero_step(j, _):
      dr_v[pl.ds(j * _L, _L)] = jnp.zeros((_L,), jnp.float32)
      return 0
    lax.fori_loop(0, _DRAIN // _L, zero_step, 0)

    @pl.when(sid < _NS - 1)
    def _():
      pltpu.sync_copy(dr_v, acc.at[pl.ds(sid * _DRAIN, _DRAIN)])

    @pl.when(sid == _NS - 1)
    def _():
      pltpu.sync_copy(dr_v.at[pl.ds(0, last)],
                      acc.at[pl.ds(15 * _DRAIN, last)])

    plsc.subcore_barrier()

    iota = lax.iota(jnp.int32, _L)

    def chunk_step(k, _):
      base = wid * epw + k * _CHUNK
      pltpu.sync_copy(conn_hbm.at[pl.ds(2 * base, 2 * _CHUNK)], conn_v)
      pltpu.sync_copy(pe_hbm.at[pl.ds(base, _CHUNK)], pe_v)
      pltpu.sync_copy(pa_hbm.at[pl.ds(base, _CHUNK)], pa_v)
      pltpu.sync_copy(pi_hbm.at[pl.ds(base, _CHUNK)], pi_v)

      def deint(i, _):
        pos = i * (2 * _L) + 2 * iota
        a = plsc.load_gather(conn_v, [pos])
        b = plsc.load_gather(conn_v, [pos + 1])
        sl = pl.ds(i * _L, _L)
        idxa_v[sl] = a
        idxb_v[sl] = b
        a3 = a + a + a
        b3 = b + b + b
        ia0[sl] = a3
        ia1[sl] = a3 + 1
        ia2[sl] = a3 + 2
        ib0[sl] = b3
        ib1[sl] = b3 + 1
        ib2[sl] = b3 + 2
        return 0
      lax.fori_loop(0, _SUB, deint, 0)

      cps = (
          (tab_cx, idxa_v, g_cxa), (tab_cz, idxa_v, g_cza),
          (tab_cx, idxb_v, g_cxb), (tab_cz, idxb_v, g_czb),
          (tab_p0, idxa_v, g_p0a), (tab_p1, idxa_v, g_p1a),
          (tab_p2, idxa_v, g_p2a), (tab_p0, idxb_v, g_p0b),
          (tab_p1, idxb_v, g_p1b), (tab_p2, idxb_v, g_p2b),
      )
      descs = [pltpu.async_copy(t.at[ix], g, gsem) for t, ix, g in cps]
      for d in descs:
        d.wait()

      def comp(i, _):
        sl = pl.ds(i * _L, _L)
        cxa = g_cxa[sl]
        cza = g_cza[sl]
        cxb = g_cxb[sl]
        czb = g_czb[sl]
        p0a = g_p0a[sl]
        p1a = g_p1a[sl]
        p2a = g_p2a[sl]
        p0b = g_p0b[sl]
        p1b = g_p1b[sl]
        p2b = g_p2b[sl]
        pe = pe_v[sl]
        pa = pa_v[sl]
        pi = pi_v[sl]

        dx = cxb - cxa
        dz = czb - cza
        q = dx * dx + dz * dz + jnp.float32(1e-10)
        r = _rsqrt(q)
        l0 = q * r
        eps = jnp.float32(1e-10)
        inv1 = jnp.float32(1.0) / (l0 + eps)
        l02 = l0 * l0
        inv2 = jnp.float32(1.0) / (l02 + eps)
        inv3 = jnp.float32(1.0) / (l02 * l0 + eps)
        c = dx * inv1
        s = dz * inv1
        ea = pe * pa
        ei = pe * pi
        k_ax = ea * inv1
        k_bend = ei * inv1
        k_sw = ei * inv2
        k_tr = ei * inv3
        ta = -p2a
        tb = -p2b
        ua = c * p0a + s * p1a
        wa = c * p1a - s * p0a
        ub = c * p0b + s * p1b
        wb = c * p1b - s * p0b
        du = ua - ub
        dw = wa - wb
        tsum = ta + tb
        f0 = k_ax * du
        f1 = jnp.float32(12.0) * k_tr * dw + jnp.float32(6.0) * k_sw * tsum
        sw6 = jnp.float32(6.0) * k_sw * dw
        f2 = sw6 + k_bend * (jnp.float32(4.0) * ta + jnp.float32(2.0) * tb)
        f5 = sw6 + k_bend * (jnp.float32(2.0) * ta + jnp.float32(4.0) * tb)
        fxa = c * f0 - s * f1
        fya = s * f0 + c * f1
        f_xa[sl] = fxa
        f_ya[sl] = fya
        f_za[sl] = f2
        f_xb[sl] = -fxa
        f_yb[sl] = -fya
        f_zb[sl] = f5
        return 0
      lax.fori_loop(0, _SUB, comp, 0)

      sps = ((f_xa, ia0), (f_ya, ia1), (f_za, ia2),
             (f_xb, ib0), (f_yb, ib1), (f_zb, ib2))
      sdescs = [pltpu.async_copy(f, acc.at[ix], ssem, add=True)
                for f, ix in sps]
      for d in sdescs:
        d.wait()
      return 0

    lax.fori_loop(0, n_chunks, chunk_step, 0)

    plsc.subcore_barrier()

    @pl.when(sid < _NS - 1)
    def _():
      pltpu.sync_copy(acc.at[pl.ds(sid * _DRAIN, _DRAIN)], dr_v)
      pltpu.sync_copy(dr_v, out_hbm.at[cid, pl.ds(sid * _DRAIN, _DRAIN)])

    @pl.when(sid == _NS - 1)
    def _():
      pltpu.sync_copy(acc.at[pl.ds(15 * _DRAIN, last)],
                      dr_v.at[pl.ds(0, last)])
      pltpu.sync_copy(dr_v.at[pl.ds(0, last)],
                      out_hbm.at[cid, pl.ds(15 * _DRAIN, last)])

  n3p = n3 + 3104
  return pl.kernel(
      body,
      out_type=jax.ShapeDtypeStruct((_NC, n3p), jnp.float32),
      mesh=mesh,
      compiler_params=pltpu.CompilerParams(use_tc_tiling_on_sc=False,
                                           needs_layout_passes=False),
      scratch_types=[
          pltpu.VMEM_SHARED((n_nodes,), jnp.float32),
          pltpu.VMEM_SHARED((n_nodes,), jnp.float32),
          pltpu.VMEM_SHARED((n_nodes,), jnp.float32),
          pltpu.VMEM_SHARED((n_nodes,), jnp.float32),
          pltpu.VMEM_SHARED((n_nodes,), jnp.float32),
          pltpu.VMEM_SHARED((n3,), jnp.float32),
          pltpu.VMEM((2 * _CHUNK,), jnp.int32),
          pltpu.VMEM((_CHUNK,), jnp.int32),
          pltpu.VMEM((_CHUNK,), jnp.int32),
          pltpu.VMEM((_CHUNK,), jnp.int32),
          pltpu.VMEM((_CHUNK,), jnp.int32),
          pltpu.VMEM((_CHUNK,), jnp.int32),
          pltpu.VMEM((_CHUNK,), jnp.int32),
          pltpu.VMEM((_CHUNK,), jnp.int32),
          pltpu.VMEM((_CHUNK,), jnp.int32),
          pltpu.VMEM((_CHUNK,), jnp.float32),
          pltpu.VMEM((_CHUNK,), jnp.float32),
          pltpu.VMEM((_CHUNK,), jnp.float32),
          pltpu.VMEM((_CHUNK,), jnp.float32),
          pltpu.VMEM((_CHUNK,), jnp.float32),
          pltpu.VMEM((_CHUNK,), jnp.float32),
          pltpu.VMEM((_CHUNK,), jnp.float32),
          pltpu.VMEM((_CHUNK,), jnp.float32),
          pltpu.VMEM((_CHUNK,), jnp.float32),
          pltpu.VMEM((_CHUNK,), jnp.float32),
          pltpu.VMEM((_CHUNK,), jnp.float32),
          pltpu.VMEM((_CHUNK,), jnp.float32),
          pltpu.VMEM((_CHUNK,), jnp.float32),
          pltpu.VMEM((_CHUNK,), jnp.float32),
          pltpu.VMEM((_CHUNK,), jnp.float32),
          pltpu.VMEM((_CHUNK,), jnp.float32),
          pltpu.VMEM((_CHUNK,), jnp.float32),
          pltpu.VMEM((_CHUNK,), jnp.float32),
          pltpu.VMEM((_CHUNK,), jnp.float32),
          pltpu.VMEM((_DRAIN,), jnp.float32),
          pltpu.SemaphoreType.DMA,
          pltpu.SemaphoreType.DMA,
      ],
  )


def _tc_combine(n3p):
  blk = n3p // 8
  assert blk % 1024 == 0

  def body(p_ref, f_ref):
    f_ref[...] = p_ref[0] + p_ref[1]

  return pl.pallas_call(
      body,
      grid=(8,),
      in_specs=[pl.BlockSpec((_NC, blk), lambda j: (0, j))],
      out_specs=pl.BlockSpec((blk,), lambda j: (j,)),
      out_shape=jax.ShapeDtypeStruct((n3p,), jnp.float32),
  )


def _tc_phys(n_nodes):
  blk = n_nodes // 10

  def body(pred_ref, sc_ref, ph_ref):
    ph_ref[...] = pred_ref[...] * sc_ref[...]

  return pl.pallas_call(
      body,
      grid=(10,),
      in_specs=[
          pl.BlockSpec((blk, 4), lambda j: (j, 0)),
          pl.BlockSpec((1, 4), lambda j: (0, 0)),
      ],
      out_specs=pl.BlockSpec((blk, 4), lambda j: (j, 0)),
      out_shape=jax.ShapeDtypeStruct((n_nodes, 4), jnp.float32),
  )


def kernel(pred_norm, connectivity, coords_norm, prop_E_norm, prop_A_norm,
           prop_I22_norm, F_ext_norm, u_scale, theta_scale):
  n_nodes = pred_norm.shape[0]
  n_edges = connectivity.shape[0]

  conn_flat = connectivity.reshape(-1)
  cols = jnp.stack([coords_norm[:, 0], coords_norm[:, 2],
                    pred_norm[:, 0], pred_norm[:, 1], pred_norm[:, 2]])
  partials = _sc_edge_kernel(n_nodes, n_edges)(
      conn_flat, prop_E_norm, prop_A_norm, prop_I22_norm, cols)

  n3 = 3 * n_nodes
  forces_flat = _tc_combine(partials.shape[1])(partials)
  forces = forces_flat[:n3].reshape(n_nodes, 3)

  scales = jnp.concatenate(
      [u_scale, u_scale, theta_scale, jnp.zeros((1,), jnp.float32)]
  ).reshape(1, 4)
  pred4 = jnp.pad(pred_norm, ((0, 0), (0, 1)))
  phys_disp = _tc_phys(n_nodes)(pred4, scales)[:, :3]
  return (forces, F_ext_norm, phys_disp)


# speedup vs baseline: 547.9079x; 5.7359x over previous
import functools

import jax
import jax.numpy as jnp
from jax import lax
from jax.experimental import pallas as pl
from jax.experimental.pallas import tpu as pltpu
from jax.experimental.pallas import tpu_sc as plsc

_NC = 2
_NS = 16
_L = 16

_CHUNK = 2000
_SUB = _CHUNK // _L

_DRAIN = 18768


def _rsqrt(q):
  qi = lax.bitcast_convert_type(q, jnp.int32)
  yi = jnp.int32(0x5F3759DF) - (qi >> 1)
  y = lax.bitcast_convert_type(yi, jnp.float32)
  h = q * jnp.float32(0.5)
  y = y * (jnp.float32(1.5) - h * y * y)
  y = y * (jnp.float32(1.5) - h * y * y)
  return y


def _sc_edge_kernel(n_nodes, n_edges):
  epw = n_edges // (_NC * _NS)
  n_chunks = epw // _CHUNK
  n3 = 3 * n_nodes
  last = n3 - 15 * _DRAIN
  stage = 6256
  stage_last = n_nodes - 15 * stage

  mesh = plsc.VectorSubcoreMesh(core_axis_name="c", subcore_axis_name="s")

  def body(na_hbm, nb_hbm, pe_hbm, pa_hbm, pi_hbm, cx_hbm, cz_hbm, p0_hbm,
           p1_hbm, p2_hbm, out0_hbm, out1_hbm,
           tab_cx, tab_cz, tab_p0, tab_p1, tab_p2, acc,
           idxa_v, idxb_v, ia0, ia1, ia2, ib0, ib1, ib2,
           pe_v, pa_v, pi_v,
           g_cxa, g_cza, g_cxb, g_czb, g_p0a, g_p1a, g_p2a, g_p0b, g_p1b,
           g_p2b, f_xa, f_ya, f_za, f_xb, f_yb, f_zb, dr_v,
           gsem, ssem):
    cid = lax.axis_index("c")
    sid = lax.axis_index("s")
    wid = cid * _NS + sid
    tabs = (tab_cx, tab_cz, tab_p0, tab_p1, tab_p2)

    cols = (cx_hbm, cz_hbm, p0_hbm, p1_hbm, p2_hbm)

    def stage_tab(src, ref, cnt, off):
      pltpu.sync_copy(src.at[pl.ds(off, cnt)], dr_v.at[pl.ds(0, cnt)])
      pltpu.sync_copy(dr_v.at[pl.ds(0, cnt)], ref.at[pl.ds(off, cnt)])

    @pl.when(sid < _NS - 1)
    def _():
      for src, ref in zip(cols, tabs):
        stage_tab(src, ref, stage, pl.multiple_of(sid * stage, 8))

    @pl.when(sid == _NS - 1)
    def _():
      for src, ref in zip(cols, tabs):
        stage_tab(src, ref, stage_last, 15 * stage)

    def zero_step(j, _):
      dr_v[pl.ds(j * _L, _L)] = jnp.zeros((_L,), jnp.float32)
      return 0
    lax.fori_loop(0, _DRAIN // _L, zero_step, 0)

    @pl.when(sid < _NS - 1)
    def _():
      pltpu.sync_copy(dr_v, acc.at[pl.ds(pl.multiple_of(sid * _DRAIN, 8),
                                          _DRAIN)])

    @pl.when(sid == _NS - 1)
    def _():
      pltpu.sync_copy(dr_v.at[pl.ds(0, last)],
                      acc.at[pl.ds(15 * _DRAIN, last)])

    plsc.subcore_barrier()

    iota = lax.iota(jnp.int32, _L)

    def chunk_step(k, _):
      base = pl.multiple_of(wid * epw + k * _CHUNK, 8)
      pltpu.sync_copy(na_hbm.at[pl.ds(base, _CHUNK)], idxa_v)
      pltpu.sync_copy(nb_hbm.at[pl.ds(base, _CHUNK)], idxb_v)
      pltpu.sync_copy(pe_hbm.at[pl.ds(base, _CHUNK)], pe_v)
      pltpu.sync_copy(pa_hbm.at[pl.ds(base, _CHUNK)], pa_v)
      pltpu.sync_copy(pi_hbm.at[pl.ds(base, _CHUNK)], pi_v)

      def deint(i, _):
        sl = pl.ds(i * _L, _L)
        a3 = idxa_v[sl] * 3
        b3 = idxb_v[sl] * 3
        ia0[sl] = a3
        ia1[sl] = a3 + 1
        ia2[sl] = a3 + 2
        ib0[sl] = b3
        ib1[sl] = b3 + 1
        ib2[sl] = b3 + 2
        return 0
      lax.fori_loop(0, _SUB, deint, 0)

      cps = (
          (tab_cx, idxa_v, g_cxa), (tab_cz, idxa_v, g_cza),
          (tab_cx, idxb_v, g_cxb), (tab_cz, idxb_v, g_czb),
          (tab_p0, idxa_v, g_p0a), (tab_p1, idxa_v, g_p1a),
          (tab_p2, idxa_v, g_p2a), (tab_p0, idxb_v, g_p0b),
          (tab_p1, idxb_v, g_p1b), (tab_p2, idxb_v, g_p2b),
      )
      descs = [pltpu.async_copy(t.at[ix], g, gsem) for t, ix, g in cps]
      for d in descs:
        d.wait()

      def comp(i, _):
        sl = pl.ds(i * _L, _L)
        cxa = g_cxa[sl]
        cza = g_cza[sl]
        cxb = g_cxb[sl]
        czb = g_czb[sl]
        p0a = g_p0a[sl]
        p1a = g_p1a[sl]
        p2a = g_p2a[sl]
        p0b = g_p0b[sl]
        p1b = g_p1b[sl]
        p2b = g_p2b[sl]
        pe = pe_v[sl]
        pa = pa_v[sl]
        pi = pi_v[sl]

        dx = cxb - cxa
        dz = czb - cza
        q = dx * dx + dz * dz + jnp.float32(1e-10)
        r = _rsqrt(q)
        l0 = q * r
        eps = jnp.float32(1e-10)
        inv1 = jnp.float32(1.0) / (l0 + eps)
        l02 = l0 * l0
        inv2 = jnp.float32(1.0) / (l02 + eps)
        inv3 = jnp.float32(1.0) / (l02 * l0 + eps)
        c = dx * inv1
        s = dz * inv1
        ea = pe * pa
        ei = pe * pi
        k_ax = ea * inv1
        k_bend = ei * inv1
        k_sw = ei * inv2
        k_tr = ei * inv3
        ta = -p2a
        tb = -p2b
        ua = c * p0a + s * p1a
        wa = c * p1a - s * p0a
        ub = c * p0b + s * p1b
        wb = c * p1b - s * p0b
        du = ua - ub
        dw = wa - wb
        tsum = ta + tb
        f0 = k_ax * du
        f1 = jnp.float32(12.0) * k_tr * dw + jnp.float32(6.0) * k_sw * tsum
        sw6 = jnp.float32(6.0) * k_sw * dw
        f2 = sw6 + k_bend * (jnp.float32(4.0) * ta + jnp.float32(2.0) * tb)
        f5 = sw6 + k_bend * (jnp.float32(2.0) * ta + jnp.float32(4.0) * tb)
        fxa = c * f0 - s * f1
        fya = s * f0 + c * f1
        f_xa[sl] = fxa
        f_ya[sl] = fya
        f_za[sl] = f2
        f_xb[sl] = -fxa
        f_yb[sl] = -fya
        f_zb[sl] = f5
        return 0
      lax.fori_loop(0, _SUB, comp, 0)

      sps = ((f_xa, ia0), (f_ya, ia1), (f_za, ia2),
             (f_xb, ib0), (f_yb, ib1), (f_zb, ib2))
      sdescs = [pltpu.async_copy(f, acc.at[ix], ssem, add=True)
                for f, ix in sps]
      for d in sdescs:
        d.wait()
      return 0

    lax.fori_loop(0, n_chunks, chunk_step, 0)

    plsc.subcore_barrier()

    for c, o_hbm in ((0, out0_hbm), (1, out1_hbm)):
      @pl.when(jnp.logical_and(cid == c, sid < _NS - 1))
      def _():
        off = pl.multiple_of(sid * _DRAIN, 8)
        pltpu.sync_copy(acc.at[pl.ds(off, _DRAIN)], dr_v)
        pltpu.sync_copy(dr_v, o_hbm.at[pl.ds(off, _DRAIN)])

      @pl.when(jnp.logical_and(cid == c, sid == _NS - 1))
      def _():
        pltpu.sync_copy(acc.at[pl.ds(15 * _DRAIN, last)],
                        dr_v.at[pl.ds(0, last)])
        pltpu.sync_copy(dr_v.at[pl.ds(0, last)],
                        o_hbm.at[pl.ds(15 * _DRAIN, last)])

  n3p = n3 + 3104
  return pl.kernel(
      body,
      out_type=(jax.ShapeDtypeStruct((n3p,), jnp.float32),
                jax.ShapeDtypeStruct((n3p,), jnp.float32)),
      mesh=mesh,
      compiler_params=pltpu.CompilerParams(needs_layout_passes=False),
      scratch_types=[
          pltpu.VMEM_SHARED((n_nodes,), jnp.float32),
          pltpu.VMEM_SHARED((n_nodes,), jnp.float32),
          pltpu.VMEM_SHARED((n_nodes,), jnp.float32),
          pltpu.VMEM_SHARED((n_nodes,), jnp.float32),
          pltpu.VMEM_SHARED((n_nodes,), jnp.float32),
          pltpu.VMEM_SHARED((n3,), jnp.float32),
          pltpu.VMEM((_CHUNK,), jnp.int32),
          pltpu.VMEM((_CHUNK,), jnp.int32),
          pltpu.VMEM((_CHUNK,), jnp.int32),
          pltpu.VMEM((_CHUNK,), jnp.int32),
          pltpu.VMEM((_CHUNK,), jnp.int32),
          pltpu.VMEM((_CHUNK,), jnp.int32),
          pltpu.VMEM((_CHUNK,), jnp.int32),
          pltpu.VMEM((_CHUNK,), jnp.int32),
          pltpu.VMEM((_CHUNK,), jnp.float32),
          pltpu.VMEM((_CHUNK,), jnp.float32),
          pltpu.VMEM((_CHUNK,), jnp.float32),
          pltpu.VMEM((_CHUNK,), jnp.float32),
          pltpu.VMEM((_CHUNK,), jnp.float32),
          pltpu.VMEM((_CHUNK,), jnp.float32),
          pltpu.VMEM((_CHUNK,), jnp.float32),
          pltpu.VMEM((_CHUNK,), jnp.float32),
          pltpu.VMEM((_CHUNK,), jnp.float32),
          pltpu.VMEM((_CHUNK,), jnp.float32),
          pltpu.VMEM((_CHUNK,), jnp.float32),
          pltpu.VMEM((_CHUNK,), jnp.float32),
          pltpu.VMEM((_CHUNK,), jnp.float32),
          pltpu.VMEM((_CHUNK,), jnp.float32),
          pltpu.VMEM((_CHUNK,), jnp.float32),
          pltpu.VMEM((_CHUNK,), jnp.float32),
          pltpu.VMEM((_CHUNK,), jnp.float32),
          pltpu.VMEM((_CHUNK,), jnp.float32),
          pltpu.VMEM((_CHUNK,), jnp.float32),
          pltpu.VMEM((_DRAIN,), jnp.float32),
          pltpu.SemaphoreType.DMA,
          pltpu.SemaphoreType.DMA,
      ],
  )


def _tc_combine(n3p):
  blk = n3p // 8
  assert blk % 1024 == 0

  def body(p0_ref, p1_ref, f_ref):
    f_ref[...] = p0_ref[...] + p1_ref[...]

  return pl.pallas_call(
      body,
      grid=(8,),
      in_specs=[pl.BlockSpec((blk,), lambda j: (j,)),
                pl.BlockSpec((blk,), lambda j: (j,))],
      out_specs=pl.BlockSpec((blk,), lambda j: (j,)),
      out_shape=jax.ShapeDtypeStruct((n3p,), jnp.float32),
  )


def _tc_phys(n_nodes):
  blk = n_nodes // 10

  def body(pred_ref, sc_ref, ph_ref):
    ph_ref[...] = pred_ref[...] * sc_ref[...]

  return pl.pallas_call(
      body,
      grid=(10,),
      in_specs=[
          pl.BlockSpec((blk, 4), lambda j: (j, 0)),
          pl.BlockSpec((1, 4), lambda j: (0, 0)),
      ],
      out_specs=pl.BlockSpec((blk, 4), lambda j: (j, 0)),
      out_shape=jax.ShapeDtypeStruct((n_nodes, 4), jnp.float32),
  )


def kernel(pred_norm, connectivity, coords_norm, prop_E_norm, prop_A_norm,
           prop_I22_norm, F_ext_norm, u_scale, theta_scale):
  n_nodes = pred_norm.shape[0]
  n_edges = connectivity.shape[0]

  part0, part1 = _sc_edge_kernel(n_nodes, n_edges)(
      connectivity[:, 0], connectivity[:, 1],
      prop_E_norm, prop_A_norm, prop_I22_norm,
      coords_norm[:, 0], coords_norm[:, 2],
      pred_norm[:, 0], pred_norm[:, 1], pred_norm[:, 2])

  n3 = 3 * n_nodes
  forces_flat = _tc_combine(part0.shape[0])(part0, part1)
  forces = forces_flat[:n3].reshape(n_nodes, 3)

  scales = jnp.concatenate(
      [u_scale, u_scale, theta_scale, jnp.zeros((1,), jnp.float32)]
  ).reshape(1, 4)
  pred4 = jnp.pad(pred_norm, ((0, 0), (0, 1)))
  phys_disp = _tc_phys(n_nodes)(pred4, scales)[:, :3]
  return (forces, F_ext_norm, phys_disp)


# speedup vs baseline: 597.5937x; 1.0907x over previous
import functools

import jax
import jax.numpy as jnp
from jax import lax
from jax.experimental import pallas as pl
from jax.experimental.pallas import tpu as pltpu
from jax.experimental.pallas import tpu_sc as plsc

_NC = 2
_NS = 16
_L = 16

_CHUNK = 2000
_SUB = _CHUNK // _L

_DRAINB = 6256


def _rsqrt(q):
  qi = lax.bitcast_convert_type(q, jnp.int32)
  yi = jnp.int32(0x5F3759DF) - (qi >> 1)
  y = lax.bitcast_convert_type(yi, jnp.float32)
  h = q * jnp.float32(0.5)
  y = y * (jnp.float32(1.5) - h * y * y)
  y = y * (jnp.float32(1.5) - h * y * y)
  return y


def _sc_edge_kernel(n_nodes, n_edges):
  epw = n_edges // (_NC * _NS)
  n_chunks = epw // _CHUNK
  n3 = 3 * n_nodes
  stage = 6256
  stage_last = n_nodes - 15 * stage

  mesh = plsc.VectorSubcoreMesh(core_axis_name="c", subcore_axis_name="s")

  def body(na_hbm, nb_hbm, pe_hbm, pa_hbm, pi_hbm, cx_hbm, cz_hbm, p0_hbm,
           p1_hbm, p2_hbm, out0_hbm, out1_hbm,
           tab_cx, tab_cz, tab_p0, tab_p1, tab_p2, acc_x, acc_y, acc_z,
           idxa_v, idxb_v, pe_v, pa_v, pi_v,
           g_cxa, g_cza, g_cxb, g_czb, g_p0a, g_p1a, g_p2a, g_p0b, g_p1b,
           g_p2b, f_xa, f_ya, f_za, f_xb, f_yb, f_zb, dr_v,
           gsem, ssem):
    cid = lax.axis_index("c")
    sid = lax.axis_index("s")
    wid = cid * _NS + sid
    tabs = (tab_cx, tab_cz, tab_p0, tab_p1, tab_p2)

    cols = (cx_hbm, cz_hbm, p0_hbm, p1_hbm, p2_hbm)

    def stage_tab(src, ref, cnt, off):
      pltpu.sync_copy(src.at[pl.ds(off, cnt)], dr_v.at[pl.ds(0, cnt)])
      pltpu.sync_copy(dr_v.at[pl.ds(0, cnt)], ref.at[pl.ds(off, cnt)])

    @pl.when(sid < _NS - 1)
    def _():
      for src, ref in zip(cols, tabs):
        stage_tab(src, ref, stage, pl.multiple_of(sid * stage, 8))

    @pl.when(sid == _NS - 1)
    def _():
      for src, ref in zip(cols, tabs):
        stage_tab(src, ref, stage_last, 15 * stage)

    accs = (acc_x, acc_y, acc_z)

    def zero_step(j, _):
      dr_v[pl.ds(j * _L, _L)] = jnp.zeros((_L,), jnp.float32)
      return 0
    lax.fori_loop(0, _DRAINB // _L, zero_step, 0)

    @pl.when(sid < _NS - 1)
    def _():
      for a in accs:
        pltpu.sync_copy(dr_v.at[pl.ds(0, stage)],
                        a.at[pl.ds(pl.multiple_of(sid * stage, 8), stage)])

    @pl.when(sid == _NS - 1)
    def _():
      for a in accs:
        pltpu.sync_copy(dr_v.at[pl.ds(0, stage_last)],
                        a.at[pl.ds(15 * stage, stage_last)])

    plsc.subcore_barrier()

    def chunk_step(k, _):
      base = pl.multiple_of(wid * epw + k * _CHUNK, 8)
      pltpu.sync_copy(na_hbm.at[pl.ds(base, _CHUNK)], idxa_v)
      pltpu.sync_copy(nb_hbm.at[pl.ds(base, _CHUNK)], idxb_v)
      pltpu.sync_copy(pe_hbm.at[pl.ds(base, _CHUNK)], pe_v)
      pltpu.sync_copy(pa_hbm.at[pl.ds(base, _CHUNK)], pa_v)
      pltpu.sync_copy(pi_hbm.at[pl.ds(base, _CHUNK)], pi_v)

      cps = (
          (tab_cx, idxa_v, g_cxa), (tab_cz, idxa_v, g_cza),
          (tab_cx, idxb_v, g_cxb), (tab_cz, idxb_v, g_czb),
          (tab_p0, idxa_v, g_p0a), (tab_p1, idxa_v, g_p1a),
          (tab_p2, idxa_v, g_p2a), (tab_p0, idxb_v, g_p0b),
          (tab_p1, idxb_v, g_p1b), (tab_p2, idxb_v, g_p2b),
      )
      descs = [pltpu.async_copy(t.at[ix], g, gsem) for t, ix, g in cps]
      for d in descs:
        d.wait()

      def comp(i, _):
        sl = pl.ds(i * _L, _L)
        cxa = g_cxa[sl]
        cza = g_cza[sl]
        cxb = g_cxb[sl]
        czb = g_czb[sl]
        p0a = g_p0a[sl]
        p1a = g_p1a[sl]
        p2a = g_p2a[sl]
        p0b = g_p0b[sl]
        p1b = g_p1b[sl]
        p2b = g_p2b[sl]
        pe = pe_v[sl]
        pa = pa_v[sl]
        pi = pi_v[sl]

        dx = cxb - cxa
        dz = czb - cza
        q = dx * dx + dz * dz + jnp.float32(1e-10)
        r = _rsqrt(q)
        l0 = q * r
        eps = jnp.float32(1e-10)
        inv1 = jnp.float32(1.0) / (l0 + eps)
        l02 = l0 * l0
        inv2 = jnp.float32(1.0) / (l02 + eps)
        inv3 = jnp.float32(1.0) / (l02 * l0 + eps)
        c = dx * inv1
        s = dz * inv1
        ea = pe * pa
        ei = pe * pi
        k_ax = ea * inv1
        k_bend = ei * inv1
        k_sw = ei * inv2
        k_tr = ei * inv3
        ta = -p2a
        tb = -p2b
        ua = c * p0a + s * p1a
        wa = c * p1a - s * p0a
        ub = c * p0b + s * p1b
        wb = c * p1b - s * p0b
        du = ua - ub
        dw = wa - wb
        tsum = ta + tb
        f0 = k_ax * du
        f1 = jnp.float32(12.0) * k_tr * dw + jnp.float32(6.0) * k_sw * tsum
        sw6 = jnp.float32(6.0) * k_sw * dw
        f2 = sw6 + k_bend * (jnp.float32(4.0) * ta + jnp.float32(2.0) * tb)
        f5 = sw6 + k_bend * (jnp.float32(2.0) * ta + jnp.float32(4.0) * tb)
        fxa = c * f0 - s * f1
        fya = s * f0 + c * f1
        f_xa[sl] = fxa
        f_ya[sl] = fya
        f_za[sl] = f2
        f_xb[sl] = -fxa
        f_yb[sl] = -fya
        f_zb[sl] = f5
        return 0
      lax.fori_loop(0, _SUB, comp, 0)

      sps = ((f_xa, acc_x, idxa_v), (f_ya, acc_y, idxa_v),
             (f_za, acc_z, idxa_v), (f_xb, acc_x, idxb_v),
             (f_yb, acc_y, idxb_v), (f_zb, acc_z, idxb_v))
      sdescs = [pltpu.async_copy(f, a.at[ix], ssem, add=True)
                for f, a, ix in sps]
      for d in sdescs:
        d.wait()
      return 0

    lax.fori_loop(0, n_chunks, chunk_step, 0)

    plsc.subcore_barrier()

    for c, o_hbm in ((0, out0_hbm), (1, out1_hbm)):
      @pl.when(jnp.logical_and(cid == c, sid < _NS - 1))
      def _():
        off = pl.multiple_of(sid * stage, 8)
        for t, a in enumerate(accs):
          pltpu.sync_copy(a.at[pl.ds(off, stage)], dr_v.at[pl.ds(0, stage)])
          pltpu.sync_copy(dr_v.at[pl.ds(0, stage)],
                          o_hbm.at[pl.ds(t * n_nodes + off, stage)])

      @pl.when(jnp.logical_and(cid == c, sid == _NS - 1))
      def _():
        for t, a in enumerate(accs):
          pltpu.sync_copy(a.at[pl.ds(15 * stage, stage_last)],
                          dr_v.at[pl.ds(0, stage_last)])
          pltpu.sync_copy(dr_v.at[pl.ds(0, stage_last)],
                          o_hbm.at[pl.ds(t * n_nodes + 15 * stage, stage_last)])

  n3p = n3 + 3104
  return pl.kernel(
      body,
      out_type=(jax.ShapeDtypeStruct((n3p,), jnp.float32),
                jax.ShapeDtypeStruct((n3p,), jnp.float32)),
      mesh=mesh,
      compiler_params=pltpu.CompilerParams(needs_layout_passes=False),
      scratch_types=[
          pltpu.VMEM_SHARED((n_nodes,), jnp.float32),
          pltpu.VMEM_SHARED((n_nodes,), jnp.float32),
          pltpu.VMEM_SHARED((n_nodes,), jnp.float32),
          pltpu.VMEM_SHARED((n_nodes,), jnp.float32),
          pltpu.VMEM_SHARED((n_nodes,), jnp.float32),
          pltpu.VMEM_SHARED((n_nodes,), jnp.float32),
          pltpu.VMEM_SHARED((n_nodes,), jnp.float32),
          pltpu.VMEM_SHARED((n_nodes,), jnp.float32),
          pltpu.VMEM((_CHUNK,), jnp.int32),
          pltpu.VMEM((_CHUNK,), jnp.int32),
          pltpu.VMEM((_CHUNK,), jnp.float32),
          pltpu.VMEM((_CHUNK,), jnp.float32),
          pltpu.VMEM((_CHUNK,), jnp.float32),
          pltpu.VMEM((_CHUNK,), jnp.float32),
          pltpu.VMEM((_CHUNK,), jnp.float32),
          pltpu.VMEM((_CHUNK,), jnp.float32),
          pltpu.VMEM((_CHUNK,), jnp.float32),
          pltpu.VMEM((_CHUNK,), jnp.float32),
          pltpu.VMEM((_CHUNK,), jnp.float32),
          pltpu.VMEM((_CHUNK,), jnp.float32),
          pltpu.VMEM((_CHUNK,), jnp.float32),
          pltpu.VMEM((_CHUNK,), jnp.float32),
          pltpu.VMEM((_CHUNK,), jnp.float32),
          pltpu.VMEM((_CHUNK,), jnp.float32),
          pltpu.VMEM((_CHUNK,), jnp.float32),
          pltpu.VMEM((_CHUNK,), jnp.float32),
          pltpu.VMEM((_CHUNK,), jnp.float32),
          pltpu.VMEM((_CHUNK,), jnp.float32),
          pltpu.VMEM((_CHUNK,), jnp.float32),
          pltpu.VMEM((_DRAINB,), jnp.float32),
          pltpu.SemaphoreType.DMA,
          pltpu.SemaphoreType.DMA,
      ],
  )


def _tc_combine(n3p):
  blk = n3p // 8
  assert blk % 1024 == 0

  def body(p0_ref, p1_ref, f_ref):
    f_ref[...] = p0_ref[...] + p1_ref[...]

  return pl.pallas_call(
      body,
      grid=(8,),
      in_specs=[pl.BlockSpec((blk,), lambda j: (j,)),
                pl.BlockSpec((blk,), lambda j: (j,))],
      out_specs=pl.BlockSpec((blk,), lambda j: (j,)),
      out_shape=jax.ShapeDtypeStruct((n3p,), jnp.float32),
  )


def _tc_phys(n_nodes):
  blk = n_nodes // 10

  def body(pred_ref, sc_ref, ph_ref):
    ph_ref[...] = pred_ref[...] * sc_ref[...]

  return pl.pallas_call(
      body,
      grid=(10,),
      in_specs=[
          pl.BlockSpec((blk, 4), lambda j: (j, 0)),
          pl.BlockSpec((1, 4), lambda j: (0, 0)),
      ],
      out_specs=pl.BlockSpec((blk, 4), lambda j: (j, 0)),
      out_shape=jax.ShapeDtypeStruct((n_nodes, 4), jnp.float32),
  )


def kernel(pred_norm, connectivity, coords_norm, prop_E_norm, prop_A_norm,
           prop_I22_norm, F_ext_norm, u_scale, theta_scale):
  n_nodes = pred_norm.shape[0]
  n_edges = connectivity.shape[0]

  part0, part1 = _sc_edge_kernel(n_nodes, n_edges)(
      connectivity[:, 0], connectivity[:, 1],
      prop_E_norm, prop_A_norm, prop_I22_norm,
      coords_norm[:, 0], coords_norm[:, 2],
      pred_norm[:, 0], pred_norm[:, 1], pred_norm[:, 2])

  n3 = 3 * n_nodes
  forces_flat = _tc_combine(part0.shape[0])(part0, part1)
  forces = forces_flat[:n3].reshape(3, n_nodes).T

  scales = jnp.concatenate(
      [u_scale, u_scale, theta_scale, jnp.zeros((1,), jnp.float32)]
  ).reshape(1, 4)
  pred4 = jnp.pad(pred_norm, ((0, 0), (0, 1)))
  phys_disp = _tc_phys(n_nodes)(pred4, scales)[:, :3]
  return (forces, F_ext_norm, phys_disp)


# speedup vs baseline: 755.9747x; 1.2650x over previous
import functools

import jax
import jax.numpy as jnp
from jax import lax
from jax.experimental import pallas as pl
from jax.experimental.pallas import tpu as pltpu
from jax.experimental.pallas import tpu_sc as plsc

_NC = 2
_NS = 16
_L = 16

_CHUNK = 2000
_SUB = _CHUNK // _L

_STAGE = 6256
_DRAINB = 3128


def _rsqrt(q):
  qi = lax.bitcast_convert_type(q, jnp.int32)
  yi = jnp.int32(0x5F3759DF) - (qi >> 1)
  y = lax.bitcast_convert_type(yi, jnp.float32)
  h = q * jnp.float32(0.5)
  y = y * (jnp.float32(1.5) - h * y * y)
  y = y * (jnp.float32(1.5) - h * y * y)
  return y


def _sc_edge_kernel(n_nodes, n_edges):
  epw = n_edges // (_NC * _NS)
  n_chunks = epw // _CHUNK
  assert n_chunks % 2 == 0
  n3 = 3 * n_nodes
  stage_last = n_nodes - 15 * _STAGE

  mesh = plsc.VectorSubcoreMesh(core_axis_name="c", subcore_axis_name="s")

  def body(na_hbm, nb_hbm, pe_hbm, pa_hbm, pi_hbm, cx_hbm, cz_hbm, p0_hbm,
           p1_hbm, p2_hbm, out0_hbm, out1_hbm,
           tab_cx, tab_cz, tab_p0, tab_p1, tab_p2, acc_x, acc_y, acc_z,
           idxa0, idxa1, idxb0, idxb1, pe0, pe1, pa0, pa1, pi0, pi1,
           g_cxa0, g_cxa1, g_cza0, g_cza1, g_cxb0, g_cxb1, g_czb0, g_czb1,
           g_p0a0, g_p0a1, g_p1a0, g_p1a1, g_p2a0, g_p2a1,
           g_p0b0, g_p0b1, g_p1b0, g_p1b1, g_p2b0, g_p2b1,
           f_xa, f_ya, f_za, f_xb, f_yb, f_zb, dr_v,
           isem, gsem, ssem):
    cid = lax.axis_index("c")
    sid = lax.axis_index("s")
    wid = cid * _NS + sid
    tabs = (tab_cx, tab_cz, tab_p0, tab_p1, tab_p2)
    accs = (acc_x, acc_y, acc_z)

    idxa = (idxa0, idxa1)
    idxb = (idxb0, idxb1)
    pes = (pe0, pe1)
    pas = (pa0, pa1)
    pis = (pi0, pi1)
    gbufs = (
        (g_cxa0, g_cza0, g_cxb0, g_czb0, g_p0a0, g_p1a0, g_p2a0, g_p0b0,
         g_p1b0, g_p2b0),
        (g_cxa1, g_cza1, g_cxb1, g_czb1, g_p0a1, g_p1a1, g_p2a1, g_p0b1,
         g_p1b1, g_p2b1),
    )
    fbufs = (f_xa, f_ya, f_za, f_xb, f_yb, f_zb)

    cols = (cx_hbm, cz_hbm, p0_hbm, p1_hbm, p2_hbm)
    half = _STAGE // 2

    def stage_tab(srcr, ref, cnt, off):
      pltpu.sync_copy(srcr.at[pl.ds(off, cnt)], dr_v.at[pl.ds(0, cnt)])
      pltpu.sync_copy(dr_v.at[pl.ds(0, cnt)], ref.at[pl.ds(off, cnt)])

    @pl.when(sid < _NS - 1)
    def _():
      off = pl.multiple_of(sid * _STAGE, 8)
      for srcr, ref in zip(cols, tabs):
        stage_tab(srcr, ref, half, off)
        stage_tab(srcr, ref, half, off + half)

    @pl.when(sid == _NS - 1)
    def _():
      for srcr, ref in zip(cols, tabs):
        stage_tab(srcr, ref, half, 15 * _STAGE)
        stage_tab(srcr, ref, stage_last - half, 15 * _STAGE + half)

    def zero_step(j, _):
      dr_v[pl.ds(j * _L, _L)] = jnp.zeros((_L,), jnp.float32)
      return 0
    lax.fori_loop(0, _DRAINB // _L, zero_step, 0)

    @pl.when(sid < _NS - 1)
    def _():
      off = pl.multiple_of(sid * _STAGE, 8)
      for a in accs:
        pltpu.sync_copy(dr_v.at[pl.ds(0, half)], a.at[pl.ds(off, half)])
        pltpu.sync_copy(dr_v.at[pl.ds(0, half)],
                        a.at[pl.ds(off + half, half)])

    @pl.when(sid == _NS - 1)
    def _():
      for a in accs:
        pltpu.sync_copy(dr_v.at[pl.ds(0, half)],
                        a.at[pl.ds(15 * _STAGE, half)])
        pltpu.sync_copy(dr_v.at[pl.ds(0, stage_last - half)],
                        a.at[pl.ds(15 * _STAGE + half, stage_last - half)])

    plsc.subcore_barrier()

    def base_of(k):
      return pl.multiple_of(wid * epw + k * _CHUNK, 8)

    def in_descs(k, p):
      b = base_of(k)
      return (
          (na_hbm.at[pl.ds(b, _CHUNK)], idxa[p]),
          (nb_hbm.at[pl.ds(b, _CHUNK)], idxb[p]),
          (pe_hbm.at[pl.ds(b, _CHUNK)], pes[p]),
          (pa_hbm.at[pl.ds(b, _CHUNK)], pas[p]),
          (pi_hbm.at[pl.ds(b, _CHUNK)], pis[p]),
      )

    def fire_in(k, p):
      for s, d in in_descs(k, p):
        pltpu.async_copy(s, d, isem)

    def wait_in(k, p):
      for s, d in in_descs(k, p):
        pltpu.make_async_copy(s, d, isem).wait()

    def g_descs(p):
      g = gbufs[p]
      srcs = (tab_cx, tab_cz, tab_cx, tab_cz, tab_p0, tab_p1, tab_p2,
              tab_p0, tab_p1, tab_p2)
      idxs = (idxa[p], idxa[p], idxb[p], idxb[p], idxa[p], idxa[p], idxa[p],
              idxb[p], idxb[p], idxb[p])
      return tuple((t.at[ix], gg) for t, ix, gg in zip(srcs, idxs, g))

    def fire_gathers(p):
      for s, d in g_descs(p):
        pltpu.async_copy(s, d, gsem)

    def wait_gathers(p):
      for s, d in g_descs(p):
        pltpu.make_async_copy(s, d, gsem).wait()

    def s_descs(p):
      return (
          (f_xa, acc_x.at[idxa[p]]), (f_ya, acc_y.at[idxa[p]]),
          (f_za, acc_z.at[idxa[p]]), (f_xb, acc_x.at[idxb[p]]),
          (f_yb, acc_y.at[idxb[p]]), (f_zb, acc_z.at[idxb[p]]),
      )

    def fire_scatters(p):
      for s, d in s_descs(p):
        pltpu.async_copy(s, d, ssem, add=True)

    def wait_scatters(p):
      for s, d in s_descs(p):
        pltpu.make_async_copy(s, d, ssem).wait()

    def compute(p):
      g_cxa, g_cza, g_cxb, g_czb, g_p0a, g_p1a, g_p2a, g_p0b, g_p1b, \
          g_p2b = gbufs[p]
      pe_v, pa_v, pi_v = pes[p], pas[p], pis[p]

      def comp(i, _):
        sl = pl.ds(i * _L, _L)
        cxa = g_cxa[sl]
        cza = g_cza[sl]
        cxb = g_cxb[sl]
        czb = g_czb[sl]
        p0a = g_p0a[sl]
        p1a = g_p1a[sl]
        p2a = g_p2a[sl]
        p0b = g_p0b[sl]
        p1b = g_p1b[sl]
        p2b = g_p2b[sl]
        pe = pe_v[sl]
        pa = pa_v[sl]
        pi = pi_v[sl]

        dx = cxb - cxa
        dz = czb - cza
        q = dx * dx + dz * dz + jnp.float32(1e-10)
        r = _rsqrt(q)
        l0 = q * r
        eps = jnp.float32(1e-10)
        inv1 = jnp.float32(1.0) / (l0 + eps)
        l02 = l0 * l0
        inv2 = jnp.float32(1.0) / (l02 + eps)
        inv3 = jnp.float32(1.0) / (l02 * l0 + eps)
        c = dx * inv1
        s = dz * inv1
        ea = pe * pa
        ei = pe * pi
        k_ax = ea * inv1
        k_bend = ei * inv1
        k_sw = ei * inv2
        k_tr = ei * inv3
        ta = -p2a
        tb = -p2b
        ua = c * p0a + s * p1a
        wa = c * p1a - s * p0a
        ub = c * p0b + s * p1b
        wb = c * p1b - s * p0b
        du = ua - ub
        dw = wa - wb
        tsum = ta + tb
        f0 = k_ax * du
        f1 = jnp.float32(12.0) * k_tr * dw + jnp.float32(6.0) * k_sw * tsum
        sw6 = jnp.float32(6.0) * k_sw * dw
        f2 = sw6 + k_bend * (jnp.float32(4.0) * ta + jnp.float32(2.0) * tb)
        f5 = sw6 + k_bend * (jnp.float32(2.0) * ta + jnp.float32(4.0) * tb)
        fxa = c * f0 - s * f1
        fya = s * f0 + c * f1
        f_xa[sl] = fxa
        f_ya[sl] = fya
        f_za[sl] = f2
        f_xb[sl] = -fxa
        f_yb[sl] = -fya
        f_zb[sl] = f5
        return 0
      lax.fori_loop(0, _SUB, comp, 0)

    def stage_k(k, p, first=False, last=False):
      if not first:
        wait_scatters(1 - p)
      if not last:
        fire_in(k + 1, 1 - p)
      wait_gathers(p)
      compute(p)
      fire_scatters(p)
      if not last:
        wait_in(k + 1, 1 - p)
        fire_gathers(1 - p)

    fire_in(0, 0)
    wait_in(0, 0)
    fire_gathers(0)
    stage_k(0, 0, first=True)

    def pair(j, _):
      stage_k(2 * j + 1, 1)
      stage_k(2 * j + 2, 0)
      return 0
    lax.fori_loop(0, (n_chunks - 2) // 2, pair, 0)

    stage_k(n_chunks - 1, 1, last=True)
    wait_scatters(1)

    plsc.subcore_barrier()

    for c, o_hbm in ((0, out0_hbm), (1, out1_hbm)):
      @pl.when(jnp.logical_and(cid == c, sid < _NS - 1))
      def _():
        off = pl.multiple_of(sid * _STAGE, 8)
        for t, a in enumerate(accs):
          for so in (0, half):
            pltpu.sync_copy(a.at[pl.ds(off + so, half)],
                            dr_v.at[pl.ds(0, half)])
            pltpu.sync_copy(dr_v.at[pl.ds(0, half)],
                            o_hbm.at[pl.ds(t * n_nodes + off + so, half)])

      @pl.when(jnp.logical_and(cid == c, sid == _NS - 1))
      def _():
        for t, a in enumerate(accs):
          for so, cnt in ((0, half), (half, stage_last - half)):
            pltpu.sync_copy(a.at[pl.ds(15 * _STAGE + so, cnt)],
                            dr_v.at[pl.ds(0, cnt)])
            pltpu.sync_copy(
                dr_v.at[pl.ds(0, cnt)],
                o_hbm.at[pl.ds(t * n_nodes + 15 * _STAGE + so, cnt)])

  n3p = n3 + 3104
  return pl.kernel(
      body,
      out_type=(jax.ShapeDtypeStruct((n3p,), jnp.float32),
                jax.ShapeDtypeStruct((n3p,), jnp.float32)),
      mesh=mesh,
      compiler_params=pltpu.CompilerParams(needs_layout_passes=False),
      scratch_types=[
          pltpu.VMEM_SHARED((n_nodes,), jnp.float32),
          pltpu.VMEM_SHARED((n_nodes,), jnp.float32),
          pltpu.VMEM_SHARED((n_nodes,), jnp.float32),
          pltpu.VMEM_SHARED((n_nodes,), jnp.float32),
          pltpu.VMEM_SHARED((n_nodes,), jnp.float32),
          pltpu.VMEM_SHARED((n_nodes,), jnp.float32),
          pltpu.VMEM_SHARED((n_nodes,), jnp.float32),
          pltpu.VMEM_SHARED((n_nodes,), jnp.float32),
      ]
      + [pltpu.VMEM((_CHUNK,), jnp.int32) for _ in range(4)]
      + [pltpu.VMEM((_CHUNK,), jnp.float32) for _ in range(6)]
      + [pltpu.VMEM((_CHUNK,), jnp.float32) for _ in range(20)]
      + [pltpu.VMEM((_CHUNK,), jnp.float32) for _ in range(6)]
      + [
          pltpu.VMEM((_DRAINB,), jnp.float32),
          pltpu.SemaphoreType.DMA,
          pltpu.SemaphoreType.DMA,
          pltpu.SemaphoreType.DMA,
      ],
  )


def _tc_combine(n3p):
  blk = n3p // 8
  assert blk % 1024 == 0

  def body(p0_ref, p1_ref, f_ref):
    f_ref[...] = p0_ref[...] + p1_ref[...]

  return pl.pallas_call(
      body,
      grid=(8,),
      in_specs=[pl.BlockSpec((blk,), lambda j: (j,)),
                pl.BlockSpec((blk,), lambda j: (j,))],
      out_specs=pl.BlockSpec((blk,), lambda j: (j,)),
      out_shape=jax.ShapeDtypeStruct((n3p,), jnp.float32),
  )


def _tc_phys(n_nodes):
  blk = n_nodes // 10

  def body(pred_ref, sc_ref, ph_ref):
    ph_ref[...] = pred_ref[...] * sc_ref[...]

  return pl.pallas_call(
      body,
      grid=(10,),
      in_specs=[
          pl.BlockSpec((blk, 4), lambda j: (j, 0)),
          pl.BlockSpec((1, 4), lambda j: (0, 0)),
      ],
      out_specs=pl.BlockSpec((blk, 4), lambda j: (j, 0)),
      out_shape=jax.ShapeDtypeStruct((n_nodes, 4), jnp.float32),
  )


def kernel(pred_norm, connectivity, coords_norm, prop_E_norm, prop_A_norm,
           prop_I22_norm, F_ext_norm, u_scale, theta_scale):
  n_nodes = pred_norm.shape[0]
  n_edges = connectivity.shape[0]

  part0, part1 = _sc_edge_kernel(n_nodes, n_edges)(
      connectivity[:, 0], connectivity[:, 1],
      prop_E_norm, prop_A_norm, prop_I22_norm,
      coords_norm[:, 0], coords_norm[:, 2],
      pred_norm[:, 0], pred_norm[:, 1], pred_norm[:, 2])

  n3 = 3 * n_nodes
  forces_flat = _tc_combine(part0.shape[0])(part0, part1)
  forces = forces_flat[:n3].reshape(3, n_nodes).T

  scales = jnp.concatenate(
      [u_scale, u_scale, theta_scale, jnp.zeros((1,), jnp.float32)]
  ).reshape(1, 4)
  pred4 = jnp.pad(pred_norm, ((0, 0), (0, 1)))
  phys_disp = _tc_phys(n_nodes)(pred4, scales)[:, :3]
  return (forces, F_ext_norm, phys_disp)


# speedup vs baseline: 757.6207x; 1.0022x over previous
import functools

import jax
import jax.numpy as jnp
from jax import lax
from jax.experimental import pallas as pl
from jax.experimental.pallas import tpu as pltpu
from jax.experimental.pallas import tpu_sc as plsc

_NC = 2
_NS = 16
_L = 16

_CHUNK = 2000
_SUB = _CHUNK // _L

_STAGE = 6256
_DRAINB = 3128


def _rsqrt(q):
  qi = lax.bitcast_convert_type(q, jnp.int32)
  yi = jnp.int32(0x5F3759DF) - (qi >> 1)
  y = lax.bitcast_convert_type(yi, jnp.float32)
  h = q * jnp.float32(0.5)
  y = y * (jnp.float32(1.5) - h * y * y)
  y = y * (jnp.float32(1.5) - h * y * y)
  return y


def _sc_edge_kernel(n_nodes, n_edges):
  epw = n_edges // (_NC * _NS)
  n_chunks = epw // _CHUNK
  assert n_chunks % 2 == 0
  n3 = 3 * n_nodes
  stage_last = n_nodes - 15 * _STAGE

  mesh = plsc.VectorSubcoreMesh(core_axis_name="c", subcore_axis_name="s")

  def body(na_hbm, nb_hbm, pe_hbm, pa_hbm, pi_hbm, cx_hbm, cz_hbm, p0_hbm,
           p1_hbm, p2_hbm, out0_hbm, out1_hbm,
           tab_cx, tab_cz, tab_p0, tab_p1, tab_p2, acc_x, acc_y, acc_z,
           idxab0, idxab1, pe0, pe1, pa0, pa1, pi0, pi1,
           g_cx0, g_cx1, g_cz0, g_cz1, g_p00, g_p01, g_p10, g_p11,
           g_p20, g_p21, f_x, f_y, f_z, dr_v,
           isem, gsem, ssem):
    cid = lax.axis_index("c")
    sid = lax.axis_index("s")
    wid = cid * _NS + sid
    tabs = (tab_cx, tab_cz, tab_p0, tab_p1, tab_p2)
    accs = (acc_x, acc_y, acc_z)

    idxab = (idxab0, idxab1)
    pes = (pe0, pe1)
    pas = (pa0, pa1)
    pis = (pi0, pi1)
    gbufs = ((g_cx0, g_cz0, g_p00, g_p10, g_p20),
             (g_cx1, g_cz1, g_p01, g_p11, g_p21))

    cols = (cx_hbm, cz_hbm, p0_hbm, p1_hbm, p2_hbm)
    half = _STAGE // 2

    def stage_tab(srcr, ref, cnt, off):
      pltpu.sync_copy(srcr.at[pl.ds(off, cnt)], dr_v.at[pl.ds(0, cnt)])
      pltpu.sync_copy(dr_v.at[pl.ds(0, cnt)], ref.at[pl.ds(off, cnt)])

    @pl.when(sid < _NS - 1)
    def _():
      off = pl.multiple_of(sid * _STAGE, 8)
      for srcr, ref in zip(cols, tabs):
        stage_tab(srcr, ref, half, off)
        stage_tab(srcr, ref, half, off + half)

    @pl.when(sid == _NS - 1)
    def _():
      for srcr, ref in zip(cols, tabs):
        stage_tab(srcr, ref, half, 15 * _STAGE)
        stage_tab(srcr, ref, stage_last - half, 15 * _STAGE + half)

    def zero_step(j, _):
      dr_v[pl.ds(j * _L, _L)] = jnp.zeros((_L,), jnp.float32)
      return 0
    lax.fori_loop(0, _DRAINB // _L, zero_step, 0)

    @pl.when(sid < _NS - 1)
    def _():
      off = pl.multiple_of(sid * _STAGE, 8)
      for a in accs:
        pltpu.sync_copy(dr_v.at[pl.ds(0, half)], a.at[pl.ds(off, half)])
        pltpu.sync_copy(dr_v.at[pl.ds(0, half)],
                        a.at[pl.ds(off + half, half)])

    @pl.when(sid == _NS - 1)
    def _():
      for a in accs:
        pltpu.sync_copy(dr_v.at[pl.ds(0, half)],
                        a.at[pl.ds(15 * _STAGE, half)])
        pltpu.sync_copy(dr_v.at[pl.ds(0, stage_last - half)],
                        a.at[pl.ds(15 * _STAGE + half, stage_last - half)])

    plsc.subcore_barrier()

    def base_of(k):
      return pl.multiple_of(wid * epw + k * _CHUNK, 8)

    def in_descs(k, p):
      b = base_of(k)
      return (
          (na_hbm.at[pl.ds(b, _CHUNK)], idxab[p].at[pl.ds(0, _CHUNK)]),
          (nb_hbm.at[pl.ds(b, _CHUNK)], idxab[p].at[pl.ds(_CHUNK, _CHUNK)]),
          (pe_hbm.at[pl.ds(b, _CHUNK)], pes[p]),
          (pa_hbm.at[pl.ds(b, _CHUNK)], pas[p]),
          (pi_hbm.at[pl.ds(b, _CHUNK)], pis[p]),
      )

    def fire_in(k, p):
      for s, d in in_descs(k, p):
        pltpu.async_copy(s, d, isem)

    def wait_in(k, p):
      for s, d in in_descs(k, p):
        pltpu.make_async_copy(s, d, isem).wait()

    def g_descs(p):
      ix = idxab[p]
      return tuple(
          (t.at[ix], gg) for t, gg in zip(tabs, gbufs[p]))

    def fire_gathers(p):
      for s, d in g_descs(p):
        pltpu.async_copy(s, d, gsem)

    def wait_gathers(p):
      for s, d in g_descs(p):
        pltpu.make_async_copy(s, d, gsem).wait()

    def s_descs(p):
      ix = idxab[p]
      return ((f_x, acc_x.at[ix]), (f_y, acc_y.at[ix]), (f_z, acc_z.at[ix]))

    def fire_scatters(p):
      for s, d in s_descs(p):
        pltpu.async_copy(s, d, ssem, add=True)

    def wait_scatters(p):
      for s, d in s_descs(p):
        pltpu.make_async_copy(s, d, ssem).wait()

    def compute(p):
      g_cx, g_cz, g_p0, g_p1, g_p2 = gbufs[p]
      pe_v, pa_v, pi_v = pes[p], pas[p], pis[p]

      def comp(i, _):
        sl = pl.ds(i * _L, _L)
        slb = pl.ds(_CHUNK + i * _L, _L)
        cxa = g_cx[sl]
        cza = g_cz[sl]
        cxb = g_cx[slb]
        czb = g_cz[slb]
        p0a = g_p0[sl]
        p1a = g_p1[sl]
        p2a = g_p2[sl]
        p0b = g_p0[slb]
        p1b = g_p1[slb]
        p2b = g_p2[slb]
        pe = pe_v[sl]
        pa = pa_v[sl]
        pi = pi_v[sl]

        dx = cxb - cxa
        dz = czb - cza
        q = dx * dx + dz * dz + jnp.float32(1e-10)
        r = _rsqrt(q)
        l0 = q * r
        eps = jnp.float32(1e-10)
        inv1 = jnp.float32(1.0) / (l0 + eps)
        l02 = l0 * l0
        inv2 = jnp.float32(1.0) / (l02 + eps)
        inv3 = jnp.float32(1.0) / (l02 * l0 + eps)
        c = dx * inv1
        s = dz * inv1
        ea = pe * pa
        ei = pe * pi
        k_ax = ea * inv1
        k_bend = ei * inv1
        k_sw = ei * inv2
        k_tr = ei * inv3
        ta = -p2a
        tb = -p2b
        ua = c * p0a + s * p1a
        wa = c * p1a - s * p0a
        ub = c * p0b + s * p1b
        wb = c * p1b - s * p0b
        du = ua - ub
        dw = wa - wb
        tsum = ta + tb
        f0 = k_ax * du
        f1 = jnp.float32(12.0) * k_tr * dw + jnp.float32(6.0) * k_sw * tsum
        sw6 = jnp.float32(6.0) * k_sw * dw
        f2 = sw6 + k_bend * (jnp.float32(4.0) * ta + jnp.float32(2.0) * tb)
        f5 = sw6 + k_bend * (jnp.float32(2.0) * ta + jnp.float32(4.0) * tb)
        fxa = c * f0 - s * f1
        fya = s * f0 + c * f1
        f_x[sl] = fxa
        f_y[sl] = fya
        f_z[sl] = f2
        f_x[slb] = -fxa
        f_y[slb] = -fya
        f_z[slb] = f5
        return 0
      lax.fori_loop(0, _SUB, comp, 0)

    def stage_k(k, p, first=False, last=False):
      if not first:
        wait_scatters(1 - p)
      if not last:
        fire_in(k + 1, 1 - p)
      wait_gathers(p)
      compute(p)
      fire_scatters(p)
      if not last:
        wait_in(k + 1, 1 - p)
        fire_gathers(1 - p)

    fire_in(0, 0)
    wait_in(0, 0)
    fire_gathers(0)
    stage_k(0, 0, first=True)

    def pair(j, _):
      stage_k(2 * j + 1, 1)
      stage_k(2 * j + 2, 0)
      return 0
    lax.fori_loop(0, (n_chunks - 2) // 2, pair, 0)

    stage_k(n_chunks - 1, 1, last=True)
    wait_scatters(1)

    plsc.subcore_barrier()

    for c, o_hbm in ((0, out0_hbm), (1, out1_hbm)):
      @pl.when(jnp.logical_and(cid == c, sid < _NS - 1))
      def _():
        off = pl.multiple_of(sid * _STAGE, 8)
        for t, a in enumerate(accs):
          for so in (0, half):
            pltpu.sync_copy(a.at[pl.ds(off + so, half)],
                            dr_v.at[pl.ds(0, half)])
            pltpu.sync_copy(dr_v.at[pl.ds(0, half)],
                            o_hbm.at[pl.ds(t * n_nodes + off + so, half)])

      @pl.when(jnp.logical_and(cid == c, sid == _NS - 1))
      def _():
        for t, a in enumerate(accs):
          for so, cnt in ((0, half), (half, stage_last - half)):
            pltpu.sync_copy(a.at[pl.ds(15 * _STAGE + so, cnt)],
                            dr_v.at[pl.ds(0, cnt)])
            pltpu.sync_copy(
                dr_v.at[pl.ds(0, cnt)],
                o_hbm.at[pl.ds(t * n_nodes + 15 * _STAGE + so, cnt)])

  n3p = n3 + 3104
  return pl.kernel(
      body,
      out_type=(jax.ShapeDtypeStruct((n3p,), jnp.float32),
                jax.ShapeDtypeStruct((n3p,), jnp.float32)),
      mesh=mesh,
      compiler_params=pltpu.CompilerParams(needs_layout_passes=False),
      scratch_types=[
          pltpu.VMEM_SHARED((n_nodes,), jnp.float32),
          pltpu.VMEM_SHARED((n_nodes,), jnp.float32),
          pltpu.VMEM_SHARED((n_nodes,), jnp.float32),
          pltpu.VMEM_SHARED((n_nodes,), jnp.float32),
          pltpu.VMEM_SHARED((n_nodes,), jnp.float32),
          pltpu.VMEM_SHARED((n_nodes,), jnp.float32),
          pltpu.VMEM_SHARED((n_nodes,), jnp.float32),
          pltpu.VMEM_SHARED((n_nodes,), jnp.float32),
      ]
      + [pltpu.VMEM((2 * _CHUNK,), jnp.int32) for _ in range(2)]
      + [pltpu.VMEM((_CHUNK,), jnp.float32) for _ in range(6)]
      + [pltpu.VMEM((2 * _CHUNK,), jnp.float32) for _ in range(10)]
      + [pltpu.VMEM((2 * _CHUNK,), jnp.float32) for _ in range(3)]
      + [
          pltpu.VMEM((_DRAINB,), jnp.float32),
          pltpu.SemaphoreType.DMA,
          pltpu.SemaphoreType.DMA,
          pltpu.SemaphoreType.DMA,
      ],
  )


def _tc_combine(n3p):
  blk = n3p // 8
  assert blk % 1024 == 0

  def body(p0_ref, p1_ref, f_ref):
    f_ref[...] = p0_ref[...] + p1_ref[...]

  return pl.pallas_call(
      body,
      grid=(8,),
      in_specs=[pl.BlockSpec((blk,), lambda j: (j,)),
                pl.BlockSpec((blk,), lambda j: (j,))],
      out_specs=pl.BlockSpec((blk,), lambda j: (j,)),
      out_shape=jax.ShapeDtypeStruct((n3p,), jnp.float32),
  )


def _tc_phys(n_nodes):
  blk = n_nodes // 10

  def body(pred_ref, sc_ref, ph_ref):
    ph_ref[...] = pred_ref[...] * sc_ref[...]

  return pl.pallas_call(
      body,
      grid=(10,),
      in_specs=[
          pl.BlockSpec((blk, 4), lambda j: (j, 0)),
          pl.BlockSpec((1, 4), lambda j: (0, 0)),
      ],
      out_specs=pl.BlockSpec((blk, 4), lambda j: (j, 0)),
      out_shape=jax.ShapeDtypeStruct((n_nodes, 4), jnp.float32),
  )


def kernel(pred_norm, connectivity, coords_norm, prop_E_norm, prop_A_norm,
           prop_I22_norm, F_ext_norm, u_scale, theta_scale):
  n_nodes = pred_norm.shape[0]
  n_edges = connectivity.shape[0]

  part0, part1 = _sc_edge_kernel(n_nodes, n_edges)(
      connectivity[:, 0], connectivity[:, 1],
      prop_E_norm, prop_A_norm, prop_I22_norm,
      coords_norm[:, 0], coords_norm[:, 2],
      pred_norm[:, 0], pred_norm[:, 1], pred_norm[:, 2])

  n3 = 3 * n_nodes
  forces_flat = _tc_combine(part0.shape[0])(part0, part1)
  forces = forces_flat[:n3].reshape(3, n_nodes).T

  scales = jnp.concatenate(
      [u_scale, u_scale, theta_scale, jnp.zeros((1,), jnp.float32)]
  ).reshape(1, 4)
  pred4 = jnp.pad(pred_norm, ((0, 0), (0, 1)))
  phys_disp = _tc_phys(n_nodes)(pred4, scales)[:, :3]
  return (forces, F_ext_norm, phys_disp)


# speedup vs baseline: 757.6713x; 1.0001x over previous
import jax
import jax.numpy as jnp
from jax import lax
from jax.experimental import pallas as pl
from jax.experimental.pallas import tpu as pltpu
from jax.experimental.pallas import tpu_sc as plsc

_NC = 2
_NS = 16
_L = 16

_CHUNK = 2000
_SUB = _CHUNK // _L

_STAGE = 6256
_DRAINB = 3128


def _rsqrt(q):
  qi = lax.bitcast_convert_type(q, jnp.int32)
  yi = jnp.int32(0x5F3759DF) - (qi >> 1)
  y = lax.bitcast_convert_type(yi, jnp.float32)
  h = q * jnp.float32(0.5)
  y = y * (jnp.float32(1.5) - h * y * y)
  y = y * (jnp.float32(1.5) - h * y * y)
  return y


def _sc_edge_kernel(n_nodes, n_edges):
  epw = n_edges // (_NC * _NS)
  n_chunks = epw // _CHUNK
  assert n_chunks % 2 == 0
  n3 = 3 * n_nodes
  stage_last = n_nodes - 15 * _STAGE

  mesh = plsc.VectorSubcoreMesh(core_axis_name="c", subcore_axis_name="s")

  def body(na_hbm, nb_hbm, pe_hbm, pa_hbm, pi_hbm, cx_hbm, cz_hbm, p0_hbm,
           p1_hbm, p2_hbm, out0_hbm, out1_hbm,
           tab_cx, tab_cz, tab_p0, tab_p1, tab_p2, acc_x, acc_y, acc_z,
           idxab0, idxab1, pe0, pe1, pa0, pa1, pi0, pi1,
           g_cx0, g_cx1, g_cz0, g_cz1, g_p00, g_p01, g_p10, g_p11,
           g_p20, g_p21, f_x, f_y, f_z, dr_v,
           isem, gsem, ssem):
    cid = lax.axis_index("c")
    sid = lax.axis_index("s")
    wid = cid * _NS + sid
    tabs = (tab_cx, tab_cz, tab_p0, tab_p1, tab_p2)
    accs = (acc_x, acc_y, acc_z)

    idxab = (idxab0, idxab1)
    pes = (pe0, pe1)
    pas = (pa0, pa1)
    pis = (pi0, pi1)
    gbufs = ((g_cx0, g_cz0, g_p00, g_p10, g_p20),
             (g_cx1, g_cz1, g_p01, g_p11, g_p21))

    cols = (cx_hbm, cz_hbm, p0_hbm, p1_hbm, p2_hbm)
    half = _STAGE // 2

    def stage_tab(srcr, ref, cnt, off):
      pltpu.sync_copy(srcr.at[pl.ds(off, cnt)], dr_v.at[pl.ds(0, cnt)])
      pltpu.sync_copy(dr_v.at[pl.ds(0, cnt)], ref.at[pl.ds(off, cnt)])

    @pl.when(sid < _NS - 1)
    def _():
      off = pl.multiple_of(sid * _STAGE, 8)
      for srcr, ref in zip(cols, tabs):
        stage_tab(srcr, ref, half, off)
        stage_tab(srcr, ref, half, off + half)

    @pl.when(sid == _NS - 1)
    def _():
      for srcr, ref in zip(cols, tabs):
        stage_tab(srcr, ref, half, 15 * _STAGE)
        stage_tab(srcr, ref, stage_last - half, 15 * _STAGE + half)

    def zero_step(j, _):
      dr_v[pl.ds(j * _L, _L)] = jnp.zeros((_L,), jnp.float32)
      return 0
    lax.fori_loop(0, _DRAINB // _L, zero_step, 0)

    @pl.when(sid < _NS - 1)
    def _():
      off = pl.multiple_of(sid * _STAGE, 8)
      for a in accs:
        pltpu.sync_copy(dr_v.at[pl.ds(0, half)], a.at[pl.ds(off, half)])
        pltpu.sync_copy(dr_v.at[pl.ds(0, half)],
                        a.at[pl.ds(off + half, half)])

    @pl.when(sid == _NS - 1)
    def _():
      for a in accs:
        pltpu.sync_copy(dr_v.at[pl.ds(0, half)],
                        a.at[pl.ds(15 * _STAGE, half)])
        pltpu.sync_copy(dr_v.at[pl.ds(0, stage_last - half)],
                        a.at[pl.ds(15 * _STAGE + half, stage_last - half)])

    plsc.subcore_barrier()

    def base_of(k):
      return pl.multiple_of(wid * epw + k * _CHUNK, 8)

    def in_descs(k, p):
      b = base_of(k)
      return (
          (na_hbm.at[pl.ds(b, _CHUNK)], idxab[p].at[pl.ds(0, _CHUNK)]),
          (nb_hbm.at[pl.ds(b, _CHUNK)], idxab[p].at[pl.ds(_CHUNK, _CHUNK)]),
          (pe_hbm.at[pl.ds(b, _CHUNK)], pes[p]),
          (pa_hbm.at[pl.ds(b, _CHUNK)], pas[p]),
          (pi_hbm.at[pl.ds(b, _CHUNK)], pis[p]),
      )

    def fire_in(k, p):
      for s, d in in_descs(k, p):
        pltpu.async_copy(s, d, isem)

    def wait_in(k, p):
      for s, d in in_descs(k, p):
        pltpu.make_async_copy(s, d, isem).wait()

    def g_descs(p):
      ix = idxab[p]
      return tuple(
          (t.at[ix], gg) for t, gg in zip(tabs, gbufs[p]))

    def fire_gathers(p):
      for s, d in g_descs(p):
        pltpu.async_copy(s, d, gsem)

    def wait_gathers(p):
      for s, d in g_descs(p):
        pltpu.make_async_copy(s, d, gsem).wait()

    def s_descs(p):
      ix = idxab[p]
      return ((f_x, acc_x.at[ix]), (f_y, acc_y.at[ix]), (f_z, acc_z.at[ix]))

    def fire_scatters(p):
      for s, d in s_descs(p):
        pltpu.async_copy(s, d, ssem, add=True)

    def wait_scatters(p):
      for s, d in s_descs(p):
        pltpu.make_async_copy(s, d, ssem).wait()

    def compute(p):
      g_cx, g_cz, g_p0, g_p1, g_p2 = gbufs[p]
      pe_v, pa_v, pi_v = pes[p], pas[p], pis[p]

      def comp(i, _):
        sl = pl.ds(i * _L, _L)
        slb = pl.ds(_CHUNK + i * _L, _L)
        cxa = g_cx[sl]
        cza = g_cz[sl]
        cxb = g_cx[slb]
        czb = g_cz[slb]
        p0a = g_p0[sl]
        p1a = g_p1[sl]
        p2a = g_p2[sl]
        p0b = g_p0[slb]
        p1b = g_p1[slb]
        p2b = g_p2[slb]
        pe = pe_v[sl]
        pa = pa_v[sl]
        pi = pi_v[sl]

        dx = cxb - cxa
        dz = czb - cza
        q = dx * dx + dz * dz + jnp.float32(1e-10)
        r = _rsqrt(q)
        l0 = q * r
        eps = jnp.float32(1e-10)
        inv1 = jnp.float32(1.0) / (l0 + eps)
        l02 = l0 * l0
        inv2 = jnp.float32(1.0) / (l02 + eps)
        inv3 = jnp.float32(1.0) / (l02 * l0 + eps)
        c = dx * inv1
        s = dz * inv1
        ea = pe * pa
        ei = pe * pi
        k_ax = ea * inv1
        k_bend = ei * inv1
        k_sw = ei * inv2
        k_tr = ei * inv3
        ta = -p2a
        tb = -p2b
        ua = c * p0a + s * p1a
        wa = c * p1a - s * p0a
        ub = c * p0b + s * p1b
        wb = c * p1b - s * p0b
        du = ua - ub
        dw = wa - wb
        tsum = ta + tb
        f0 = k_ax * du
        f1 = jnp.float32(12.0) * k_tr * dw + jnp.float32(6.0) * k_sw * tsum
        sw6 = jnp.float32(6.0) * k_sw * dw
        f2 = sw6 + k_bend * (jnp.float32(4.0) * ta + jnp.float32(2.0) * tb)
        f5 = sw6 + k_bend * (jnp.float32(2.0) * ta + jnp.float32(4.0) * tb)
        fxa = c * f0 - s * f1
        fya = s * f0 + c * f1
        f_x[sl] = fxa
        f_y[sl] = fya
        f_z[sl] = f2
        f_x[slb] = -fxa
        f_y[slb] = -fya
        f_z[slb] = f5
        return 0
      lax.fori_loop(0, _SUB, comp, 0)

    def stage_k(k, p, first=False, last=False):
      if not first:
        wait_scatters(1 - p)
      if not last:
        fire_in(k + 1, 1 - p)
      wait_gathers(p)
      compute(p)
      fire_scatters(p)
      if not last:
        wait_in(k + 1, 1 - p)
        fire_gathers(1 - p)

    fire_in(0, 0)
    wait_in(0, 0)
    fire_gathers(0)
    stage_k(0, 0, first=True)

    def pair(j, _):
      stage_k(2 * j + 1, 1)
      stage_k(2 * j + 2, 0)
      return 0
    lax.fori_loop(0, (n_chunks - 2) // 2, pair, 0)

    stage_k(n_chunks - 1, 1, last=True)
    wait_scatters(1)

    plsc.subcore_barrier()

    for c, o_hbm in ((0, out0_hbm), (1, out1_hbm)):
      @pl.when(jnp.logical_and(cid == c, sid < _NS - 1))
      def _():
        off = pl.multiple_of(sid * _STAGE, 8)
        for t, a in enumerate(accs):
          for so in (0, half):
            pltpu.sync_copy(a.at[pl.ds(off + so, half)],
                            dr_v.at[pl.ds(0, half)])
            pltpu.sync_copy(dr_v.at[pl.ds(0, half)],
                            o_hbm.at[pl.ds(t * n_nodes + off + so, half)])

      @pl.when(jnp.logical_and(cid == c, sid == _NS - 1))
      def _():
        for t, a in enumerate(accs):
          for so, cnt in ((0, half), (half, stage_last - half)):
            pltpu.sync_copy(a.at[pl.ds(15 * _STAGE + so, cnt)],
                            dr_v.at[pl.ds(0, cnt)])
            pltpu.sync_copy(
                dr_v.at[pl.ds(0, cnt)],
                o_hbm.at[pl.ds(t * n_nodes + 15 * _STAGE + so, cnt)])

  n3p = n3 + 3104
  return pl.kernel(
      body,
      out_type=(jax.ShapeDtypeStruct((n3p,), jnp.float32),
                jax.ShapeDtypeStruct((n3p,), jnp.float32)),
      mesh=mesh,
      compiler_params=pltpu.CompilerParams(needs_layout_passes=False),
      scratch_types=[
          pltpu.VMEM_SHARED((n_nodes,), jnp.float32),
          pltpu.VMEM_SHARED((n_nodes,), jnp.float32),
          pltpu.VMEM_SHARED((n_nodes,), jnp.float32),
          pltpu.VMEM_SHARED((n_nodes,), jnp.float32),
          pltpu.VMEM_SHARED((n_nodes,), jnp.float32),
          pltpu.VMEM_SHARED((n_nodes,), jnp.float32),
          pltpu.VMEM_SHARED((n_nodes,), jnp.float32),
          pltpu.VMEM_SHARED((n_nodes,), jnp.float32),
      ]
      + [pltpu.VMEM((2 * _CHUNK,), jnp.int32) for _ in range(2)]
      + [pltpu.VMEM((_CHUNK,), jnp.float32) for _ in range(6)]
      + [pltpu.VMEM((2 * _CHUNK,), jnp.float32) for _ in range(10)]
      + [pltpu.VMEM((2 * _CHUNK,), jnp.float32) for _ in range(3)]
      + [
          pltpu.VMEM((_DRAINB,), jnp.float32),
          pltpu.SemaphoreType.DMA,
          pltpu.SemaphoreType.DMA,
          pltpu.SemaphoreType.DMA,
      ],
  )


def _tc_combine(n3p):
  blk = n3p // 8
  assert blk % 1024 == 0

  def body(p0_ref, p1_ref, f_ref):
    f_ref[...] = p0_ref[...] + p1_ref[...]

  return pl.pallas_call(
      body,
      grid=(8,),
      in_specs=[pl.BlockSpec((blk,), lambda j: (j,)),
                pl.BlockSpec((blk,), lambda j: (j,))],
      out_specs=pl.BlockSpec((blk,), lambda j: (j,)),
      out_shape=jax.ShapeDtypeStruct((n3p,), jnp.float32),
  )


def _tc_phys(n_nodes):
  blk = n_nodes // 10

  def body(pred_ref, sc_ref, ph_ref):
    ph_ref[...] = pred_ref[...] * sc_ref[...]

  return pl.pallas_call(
      body,
      grid=(10,),
      in_specs=[
          pl.BlockSpec((blk, 4), lambda j: (j, 0)),
          pl.BlockSpec((1, 4), lambda j: (0, 0)),
      ],
      out_specs=pl.BlockSpec((blk, 4), lambda j: (j, 0)),
      out_shape=jax.ShapeDtypeStruct((n_nodes, 4), jnp.float32),
  )


def kernel(pred_norm, connectivity, coords_norm, prop_E_norm, prop_A_norm,
           prop_I22_norm, F_ext_norm, u_scale, theta_scale):
  n_nodes = pred_norm.shape[0]
  n_edges = connectivity.shape[0]

  part0, part1 = _sc_edge_kernel(n_nodes, n_edges)(
      connectivity[:, 0], connectivity[:, 1],
      prop_E_norm, prop_A_norm, prop_I22_norm,
      coords_norm[:, 0], coords_norm[:, 2],
      pred_norm[:, 0], pred_norm[:, 1], pred_norm[:, 2])

  n3 = 3 * n_nodes
  forces_flat = _tc_combine(part0.shape[0])(part0, part1)
  forces = forces_flat[:n3].reshape(3, n_nodes).T

  scales = jnp.concatenate(
      [u_scale, u_scale, theta_scale, jnp.zeros((1,), jnp.float32)]
  ).reshape(1, 4)
  pred4 = jnp.pad(pred_norm, ((0, 0), (0, 1)))
  phys_disp = _tc_phys(n_nodes)(pred4, scales)[:, :3]
  return (forces, F_ext_norm, phys_disp)
